# baseline (speedup 1.0000x reference)
import jax
import jax.numpy as jnp
from jax.experimental import pallas as pl
from jax.experimental.pallas import tpu as pltpu

_ROW_BLOCK = 4096


def _body(idx_ref, emb_ref, x_ref, o_ref):
    scale = (idx_ref[0, 0, :] - 1).astype(jnp.float32)
    o_ref[...] = x_ref[...] + scale[:, None] * emb_ref[0][None, :]


def kernel(x, indices, emb_weight):
    B, S, D = x.shape
    rows = B * S
    xf = x.reshape(rows, D)
    nb = pl.cdiv(rows, _ROW_BLOCK)
    idx_flat = jnp.tile(indices, B)
    idx_pad = jnp.pad(idx_flat, (0, nb * _ROW_BLOCK - rows))
    idx3 = idx_pad.reshape(nb, 1, _ROW_BLOCK)
    out = pl.pallas_call(
        _body,
        grid=(nb,),
        in_specs=[
            pl.BlockSpec((1, 1, _ROW_BLOCK), lambda i: (i, 0, 0)),
            pl.BlockSpec((1, D), lambda i: (0, 0)),
            pl.BlockSpec((_ROW_BLOCK, D), lambda i: (i, 0)),
        ],
        out_specs=pl.BlockSpec((_ROW_BLOCK, D), lambda i: (i, 0),
                               pipeline_mode=pl.Buffered(buffer_count=1)),
        out_shape=jax.ShapeDtypeStruct((rows, D), x.dtype),
        compiler_params=pltpu.CompilerParams(
            dimension_semantics=("parallel",),
            vmem_limit_bytes=63 * 1024 * 1024,
        ),
    )(idx3, emb_weight, xf)
    return out.reshape(B, S, D)


# speedup vs baseline: 1.2161x; 1.2161x over previous
import jax
import jax.numpy as jnp
from jax.experimental import pallas as pl
from jax.experimental.pallas import tpu as pltpu

_ROW_BLOCK = 3072


def _body(idx_ref, emb_ref, x_ref, o_ref):
    scale = (idx_ref[0, 0, :] - 1).astype(jnp.float32)
    o_ref[...] = x_ref[...] + scale[:, None] * emb_ref[0][None, :]


def kernel(x, indices, emb_weight):
    B, S, D = x.shape
    rows = B * S
    xf = x.reshape(rows, D)
    nb = pl.cdiv(rows, _ROW_BLOCK)
    idx_flat = jnp.tile(indices, B)
    idx_pad = jnp.pad(idx_flat, (0, nb * _ROW_BLOCK - rows))
    idx3 = idx_pad.reshape(nb, 1, _ROW_BLOCK)
    out = pl.pallas_call(
        _body,
        grid=(nb,),
        in_specs=[
            pl.BlockSpec((1, 1, _ROW_BLOCK), lambda i: (i, 0, 0)),
            pl.BlockSpec((1, D), lambda i: (0, 0)),
            pl.BlockSpec((_ROW_BLOCK, D), lambda i: (i, 0)),
        ],
        out_specs=pl.BlockSpec((_ROW_BLOCK, D), lambda i: (i, 0)),
        out_shape=jax.ShapeDtypeStruct((rows, D), x.dtype),
        compiler_params=pltpu.CompilerParams(
            dimension_semantics=("parallel",),
            vmem_limit_bytes=63 * 1024 * 1024,
        ),
    )(idx3, emb_weight, xf)
    return out.reshape(B, S, D)


# speedup vs baseline: 1.2228x; 1.0056x over previous
import jax
import jax.numpy as jnp
from jax.experimental import pallas as pl
from jax.experimental.pallas import tpu as pltpu

_ROW_BLOCK = 3840


def _body(idx_ref, emb_ref, x_ref, o_ref):
    scale = (idx_ref[0, 0, :] - 1).astype(jnp.float32)
    o_ref[...] = x_ref[...] + scale[:, None] * emb_ref[0][None, :]


def kernel(x, indices, emb_weight):
    B, S, D = x.shape
    rows = B * S
    xf = x.reshape(rows, D)
    nb = pl.cdiv(rows, _ROW_BLOCK)
    idx_flat = jnp.tile(indices, B)
    idx_pad = jnp.pad(idx_flat, (0, nb * _ROW_BLOCK - rows))
    idx3 = idx_pad.reshape(nb, 1, _ROW_BLOCK)
    out = pl.pallas_call(
        _body,
        grid=(nb,),
        in_specs=[
            pl.BlockSpec((1, 1, _ROW_BLOCK), lambda i: (i, 0, 0)),
            pl.BlockSpec((1, D), lambda i: (0, 0)),
            pl.BlockSpec((_ROW_BLOCK, D), lambda i: (i, 0)),
        ],
        out_specs=pl.BlockSpec((_ROW_BLOCK, D), lambda i: (i, 0)),
        out_shape=jax.ShapeDtypeStruct((rows, D), x.dtype),
        compiler_params=pltpu.CompilerParams(
            dimension_semantics=("parallel",),
            vmem_limit_bytes=63 * 1024 * 1024,
        ),
    )(idx3, emb_weight, xf)
    return out.reshape(B, S, D)


# speedup vs baseline: 1.2280x; 1.0042x over previous
import jax
import jax.numpy as jnp
from jax.experimental import pallas as pl
from jax.experimental.pallas import tpu as pltpu

_ROW_BLOCK = 3840


def _body(idx_ref, emb_ref, x_ref, o_ref):
    scale = (idx_ref[0, 0, :] - 1).astype(jnp.float32)
    o_ref[...] = x_ref[...] + scale[:, None] * emb_ref[0][None, :]


def kernel(x, indices, emb_weight):
    B, S, D = x.shape
    rows = B * S
    xf = x.reshape(rows, D)
    nb = pl.cdiv(rows, _ROW_BLOCK)
    idx_flat = jnp.tile(indices, B)
    idx_pad = jnp.pad(idx_flat, (0, nb * _ROW_BLOCK - rows))
    idx3 = idx_pad.reshape(nb, 1, _ROW_BLOCK)
    out = pl.pallas_call(
        _body,
        grid=(nb,),
        in_specs=[
            pl.BlockSpec((1, 1, _ROW_BLOCK), lambda i: (i, 0, 0)),
            pl.BlockSpec((1, D), lambda i: (0, 0)),
            pl.BlockSpec((_ROW_BLOCK, D), lambda i: (i, 0)),
        ],
        out_specs=pl.BlockSpec((_ROW_BLOCK, D), lambda i: (i, 0)),
        out_shape=jax.ShapeDtypeStruct((rows, D), x.dtype),
        compiler_params=pltpu.CompilerParams(
            dimension_semantics=("parallel",),
            vmem_limit_bytes=63 * 1024 * 1024,
            allow_input_fusion=[True, False, False],
        ),
    )(idx3, emb_weight, xf)
    return out.reshape(B, S, D)
